# baseline (speedup 1.0000x reference)
import jax
import jax.numpy as jnp
from jax.experimental import pallas as pl
from jax.experimental.pallas import tpu as pltpu

_BLOCK_P = 4096


def _mlp_softmax_kernel(x_ref, w1_ref, b1_ref, w2_ref, b2_ref, out_ref, *, q):
    x = x_ref[...]
    h = jnp.dot(x, w1_ref[...], preferred_element_type=jnp.float32) + b1_ref[...]
    h = 0.5 * h * (1.0 + jax.lax.erf(h * 0.7071067811865476))
    l = jnp.dot(h, w2_ref[...], preferred_element_type=jnp.float32) + b2_ref[...]
    l0 = l[:, :q]
    l1 = l[:, q:]
    m0 = jnp.max(l0, axis=1, keepdims=True)
    m1 = jnp.max(l1, axis=1, keepdims=True)
    e0 = jnp.exp(l0 - m0)
    e1 = jnp.exp(l1 - m1)
    p0 = e0 / jnp.sum(e0, axis=1, keepdims=True)
    p1 = e1 / jnp.sum(e1, axis=1, keepdims=True)
    out_ref[...] = jnp.concatenate([p0, p1], axis=1)


import functools


def kernel(batch_x, W1, b1, W2, b2):
    n, d = batch_x.shape
    h_dim = W1.shape[1]
    q = W2.shape[1]
    dp = 2 * d
    hp = 2 * h_dim
    qp = 2 * q
    np_ = n // 2

    x2 = batch_x.reshape(np_, dp)
    zero_dh = jnp.zeros((d, h_dim), jnp.float32)
    w1p = jnp.block([[W1, zero_dh], [zero_dh, W1]])
    zero_hq = jnp.zeros((h_dim, q), jnp.float32)
    w2p = jnp.block([[W2, zero_hq], [zero_hq, W2]])
    b1p = jnp.concatenate([b1, b1]).reshape(1, hp)
    b2p = jnp.concatenate([b2, b2]).reshape(1, qp)

    grid = (np_ // _BLOCK_P,)
    out2 = pl.pallas_call(
        functools.partial(_mlp_softmax_kernel, q=q),
        grid=grid,
        in_specs=[
            pl.BlockSpec((_BLOCK_P, dp), lambda i: (i, 0)),
            pl.BlockSpec((dp, hp), lambda i: (0, 0)),
            pl.BlockSpec((1, hp), lambda i: (0, 0)),
            pl.BlockSpec((hp, qp), lambda i: (0, 0)),
            pl.BlockSpec((1, qp), lambda i: (0, 0)),
        ],
        out_specs=pl.BlockSpec((_BLOCK_P, qp), lambda i: (i, 0)),
        out_shape=jax.ShapeDtypeStruct((np_, qp), jnp.float32),
        compiler_params=pltpu.CompilerParams(
            dimension_semantics=("parallel",),
        ),
    )(x2, w1p, b1p, w2p, b2p)
    return out2.reshape(n, q)


# speedup vs baseline: 1.5699x; 1.5699x over previous
import jax
import jax.numpy as jnp
from jax.experimental import pallas as pl
from jax.experimental.pallas import tpu as pltpu

_BLOCK_N = 8192


def _mlp_softmax_kernel(x_ref, w1_ref, b1_ref, w2_ref, b2_ref, out_ref):
    x = x_ref[...]
    ht = jax.lax.dot_general(
        w1_ref[...], x, (((0,), (1,)), ((), ())),
        preferred_element_type=jnp.float32,
    ) + b1_ref[...]
    ht = 0.5 * ht * (1.0 + jax.lax.erf(ht * 0.7071067811865476))
    lt = jax.lax.dot_general(
        w2_ref[...], ht, (((0,), (0,)), ((), ())),
        preferred_element_type=jnp.float32,
    ) + b2_ref[...]
    m = jnp.max(lt, axis=0, keepdims=True)
    e = jnp.exp(lt - m)
    p = e / jnp.sum(e, axis=0, keepdims=True)
    out_ref[...] = p.T


def kernel(batch_x, W1, b1, W2, b2):
    n, d = batch_x.shape
    h_dim = W1.shape[1]
    q = W2.shape[1]
    grid = (n // _BLOCK_N,)
    return pl.pallas_call(
        _mlp_softmax_kernel,
        grid=grid,
        in_specs=[
            pl.BlockSpec((_BLOCK_N, d), lambda i: (i, 0)),
            pl.BlockSpec((d, h_dim), lambda i: (0, 0)),
            pl.BlockSpec((h_dim, 1), lambda i: (0, 0)),
            pl.BlockSpec((h_dim, q), lambda i: (0, 0)),
            pl.BlockSpec((q, 1), lambda i: (0, 0)),
        ],
        out_specs=pl.BlockSpec((_BLOCK_N, q), lambda i: (i, 0)),
        out_shape=jax.ShapeDtypeStruct((n, q), jnp.float32),
        compiler_params=pltpu.CompilerParams(
            dimension_semantics=("parallel",),
        ),
    )(batch_x, W1, b1.reshape(h_dim, 1), W2, b2.reshape(q, 1))
